# baseline (speedup 1.0000x reference)
import functools

import jax
import jax.numpy as jnp
from jax import lax
from jax.experimental import pallas as pl
from jax.experimental.pallas import tpu as pltpu
from jax.experimental.pallas import tpu_sc as plsc

_MINIMUM = 0.3
_MAXIMUM = 0.9

_NW = 32
_SPATIAL = 384 * 384
_CHUNK = _SPATIAL // _NW
_LANES = 16
_VECS = _CHUNK // _LANES
_UNROLL = 4


def _sc_stage(logits_f, masks_f, B, C, N):
    Cf = C - 1
    n_img = B * Cf
    n_rows = n_img * N

    mesh = plsc.VectorSubcoreMesh(core_axis_name="c", subcore_axis_name="s")

    @functools.partial(
        pl.kernel,
        mesh=mesh,
        out_type=jax.ShapeDtypeStruct((_NW, n_rows * 2 * _LANES), jnp.float32),
        scratch_types=[
            pltpu.VMEM((2, _CHUNK), jnp.float32),
            pltpu.VMEM((2, N, _CHUNK), jnp.float32),
            pltpu.VMEM((n_rows * 2 * _LANES,), jnp.float32),
            pltpu.SemaphoreType.DMA,
            pltpu.SemaphoreType.DMA,
            pltpu.SemaphoreType.DMA,
            pltpu.SemaphoreType.DMA,
        ],
    )
    def sc_kernel(l_hbm, m_hbm, out_hbm, lbuf, mbuf, acc,
                  sl0, sl1, sm0, sm1):
        wid = lax.axis_index("s") * 2 + lax.axis_index("c")
        base = wid * _CHUNK
        lsems = (sl0, sl1)
        msems = (sm0, sm1)

        def issue(bc, slot):
            b = bc // Cf
            c = 1 + bc % Cf
            rl = b * C + c
            cl = pltpu.async_copy(
                l_hbm.at[pl.ds(rl * _SPATIAL + base, _CHUNK)],
                lbuf.at[slot], lsems[slot])
            cm = pltpu.async_copy(
                m_hbm.at[pl.ds(rl * N, N), pl.ds(base, _CHUNK)],
                mbuf.at[slot], msems[slot])
            return cl, cm

        pend = issue(0, 0)
        for bc in range(n_img):
            slot = bc % 2
            cl, cm = pend
            if bc + 1 < n_img:
                pend = issue(bc + 1, 1 - slot)
            cl.wait()
            cm.wait()

            def v_body(k, carry):
                s1s = list(carry[:N])
                s2s = list(carry[N:])
                for u in range(_UNROLL):
                    off = (k * _UNROLL + u) * _LANES
                    lv = lbuf[slot, pl.ds(off, _LANES)]
                    for n in range(N):
                        m = mbuf[slot, n, pl.ds(off, _LANES)]
                        s1s[n] = s1s[n] + m
                        s2s[n] = s2s[n] + m * lv
                return tuple(s1s) + tuple(s2s)

            zero = jnp.zeros((_LANES,), jnp.float32)
            res = lax.fori_loop(0, _VECS // _UNROLL, v_body,
                                (zero,) * (2 * N))
            for n in range(N):
                r = bc * N + n
                acc[pl.ds(r * 2 * _LANES, _LANES)] = res[n]
                acc[pl.ds(r * 2 * _LANES + _LANES, _LANES)] = res[N + n]

        pltpu.sync_copy(acc, out_hbm.at[wid])

    return sc_kernel(logits_f, masks_f)


def _tc_finish_body(p_ref, out_ref):
    p = p_ref[...]
    box = jnp.sum(p[:, :, 0, :], axis=(0, 2))
    act = jnp.sum(p[:, :, 1, :], axis=(0, 2))
    over = act - _MAXIMUM * box
    under = _MINIMUM * box - act
    err = (jnp.where(over >= 0, over * over, 0.0)
           + jnp.where(under >= 0, under * under, 0.0))
    out_ref[0, 0] = jnp.sum(err)


def kernel(logits, box_masks):
    B, C, W, H = logits.shape
    N = box_masks.shape[2]
    Cf = C - 1

    logits_f = logits.reshape(B * C * W * H)
    masks_f = box_masks.reshape(B * C * N, W * H)

    partials = _sc_stage(logits_f, masks_f, B, C, N)
    partials = partials.reshape(_NW, B * Cf * N, 2, _LANES)

    out = pl.pallas_call(
        _tc_finish_body,
        out_specs=pl.BlockSpec(memory_space=pltpu.SMEM),
        out_shape=jax.ShapeDtypeStruct((1, 1), jnp.float32),
    )(partials)
    return out[0, 0] / float(Cf * W * H)


# speedup vs baseline: 1.7368x; 1.7368x over previous
import functools

import jax
import jax.numpy as jnp
from jax import lax
from jax.experimental import pallas as pl
from jax.experimental.pallas import tpu as pltpu
from jax.experimental.pallas import tpu_sc as plsc

_MINIMUM = 0.3
_MAXIMUM = 0.9

_NW = 32
_LANES = 16
_BAND = 16
_NBANDS = 6


def _sc_stage(logits, masks, B, C, N, W, H):
    Cf = C - 1
    kvecs = H // _LANES

    mesh = plsc.VectorSubcoreMesh(core_axis_name="c", subcore_axis_name="s")

    @functools.partial(
        pl.kernel,
        mesh=mesh,
        out_type=jax.ShapeDtypeStruct((_NW, N * 2 * _LANES), jnp.float32),
        scratch_types=[
            pltpu.VMEM((2, _BAND, H), jnp.float32),
            pltpu.VMEM((2, N, _BAND, H), jnp.float32),
            pltpu.VMEM((N * 2 * _LANES,), jnp.float32),
            pltpu.SemaphoreType.DMA,
            pltpu.SemaphoreType.DMA,
            pltpu.SemaphoreType.DMA,
            pltpu.SemaphoreType.DMA,
        ],
        compiler_params=pltpu.CompilerParams(use_tc_tiling_on_sc=True),
    )
    def sc_kernel(l_hbm, m_hbm, out_hbm, lbuf, mbuf, acc,
                  sl0, sl1, sm0, sm1):
        wid = lax.axis_index("s") * 2 + lax.axis_index("c")
        bc = wid // 4
        q = wid % 4
        b = bc // Cf
        c = 1 + bc % Cf
        r_base = q * (_BAND * _NBANDS)
        lsems = (sl0, sl1)
        msems = (sm0, sm1)

        def issue(u, slot):
            r0 = r_base + u * _BAND
            cl = pltpu.async_copy(
                l_hbm.at[b, c, pl.ds(r0, _BAND), :], lbuf.at[slot],
                lsems[slot])
            cm = pltpu.async_copy(
                m_hbm.at[b, c, :, pl.ds(r0, _BAND), :], mbuf.at[slot],
                msems[slot])
            return cl, cm

        zero = jnp.zeros((_LANES,), jnp.float32)
        accs = (zero,) * (2 * N)
        pend = issue(0, 0)
        for u in range(_NBANDS):
            slot = u % 2
            cl, cm = pend
            if u + 1 < _NBANDS:
                pend = issue(u + 1, 1 - slot)
            cl.wait()
            cm.wait()

            for r in range(_BAND):
                def v_body(k, carry, _r=r, _slot=slot):
                    s1s = list(carry[:N])
                    s2s = list(carry[N:])
                    lv = lbuf[_slot, _r, pl.ds(k * _LANES, _LANES)]
                    for n in range(N):
                        m = mbuf[_slot, n, _r, pl.ds(k * _LANES, _LANES)]
                        s1s[n] = s1s[n] + m
                        s2s[n] = s2s[n] + m * lv
                    return tuple(s1s) + tuple(s2s)

                accs = lax.fori_loop(0, kvecs, v_body, accs)

        for n in range(N):
            acc[pl.ds(n * 2 * _LANES, _LANES)] = accs[n]
            acc[pl.ds(n * 2 * _LANES + _LANES, _LANES)] = accs[N + n]
        pltpu.sync_copy(acc, out_hbm.at[wid])

    return sc_kernel(logits, masks)


def _tc_finish_body(p_ref, out_ref):
    p = p_ref[...]
    box = jnp.sum(p[:, :, :, 0, :], axis=(1, 3))
    act = jnp.sum(p[:, :, :, 1, :], axis=(1, 3))
    over = act - _MAXIMUM * box
    under = _MINIMUM * box - act
    err = (jnp.where(over >= 0, over * over, 0.0)
           + jnp.where(under >= 0, under * under, 0.0))
    out_ref[0, 0] = jnp.sum(err)


def kernel(logits, box_masks):
    B, C, W, H = logits.shape
    N = box_masks.shape[2]
    Cf = C - 1

    partials = _sc_stage(logits, box_masks, B, C, N, W, H)
    partials = partials.reshape(B * Cf, 4, N, 2, _LANES)

    out = pl.pallas_call(
        _tc_finish_body,
        out_specs=pl.BlockSpec(memory_space=pltpu.SMEM),
        out_shape=jax.ShapeDtypeStruct((1, 1), jnp.float32),
    )(partials)
    return out[0, 0] / float(Cf * W * H)
